# baseline (speedup 1.0000x reference)
import functools

import jax
import jax.numpy as jnp
from jax import lax
from jax.experimental import pallas as pl
from jax.experimental.pallas import tpu as pltpu
from jax.experimental.pallas import tpu_sc as plsc

_EMBED_DIM = 32
_NUM_CORES = 2
_NUM_SUBCORES = 16
_NUM_WORKERS = _NUM_CORES * _NUM_SUBCORES
_BPW = 128
_TC = 4


@functools.partial(jax.jit, static_argnames=("hist",))
def _sc_gather(idx_t, table, *, hist):
    batch = idx_t.shape[1]
    n_chunks = hist // _TC
    mesh = plsc.VectorSubcoreMesh(core_axis_name="c", subcore_axis_name="s")

    @functools.partial(
        pl.kernel,
        mesh=mesh,
        out_type=jax.ShapeDtypeStruct((hist, 4, _NUM_WORKERS, 8, 128),
                                      jnp.float32),
        scratch_types=[
            pltpu.VMEM((_TC, _BPW), jnp.int32),
            pltpu.VMEM((_TC * _BPW, _EMBED_DIM), jnp.float32),
            pltpu.VMEM((_TC, 4, 1, 8, 128), jnp.float32),
            pltpu.SemaphoreType.DMA,
            pltpu.SemaphoreType.DMA,
        ],
        compiler_params=pltpu.CompilerParams(use_tc_tiling_on_sc=False,
                                             needs_layout_passes=False),
    )
    def k(idx_hbm, table_hbm, out_hbm, idx_v, rows_v, tile_v, gsem, osem):
        wid = lax.axis_index("s") * _NUM_CORES + lax.axis_index("c")
        b0 = wid * _BPW
        lane = lax.iota(jnp.int32, 16)

        def chunk(j, carry):
            t0 = j * _TC
            pltpu.sync_copy(
                idx_hbm.at[pl.ds(t0, _TC), pl.ds(b0, _BPW)], idx_v)
            for tp in range(_TC):
                pltpu.async_copy(
                    table_hbm.at[idx_v.at[tp]],
                    rows_v.at[pl.ds(tp * _BPW, _BPW)], gsem)
            for tp in range(_TC):
                pltpu.make_async_copy(
                    table_hbm.at[idx_v.at[tp]],
                    rows_v.at[pl.ds(tp * _BPW, _BPW)], gsem).wait()

            def trans(m, carry2):
                tp = m // _EMBED_DIM
                e = m % _EMBED_DIM
                te = e // 8
                ei = e % 8
                col = jnp.full((16,), e, jnp.int32)
                for g in range(8):
                    row = lane + (tp * _BPW + g * 16)
                    vals = plsc.load_gather(rows_v, [row, col])
                    tile_v[tp, te, 0, ei, pl.ds(g * 16, 16)] = vals
                return carry2

            lax.fori_loop(0, _TC * _EMBED_DIM, trans, 0)
            pltpu.async_copy(
                tile_v,
                out_hbm.at[pl.ds(t0, _TC), pl.ds(0, 4), pl.ds(wid, 1),
                           pl.ds(0, 8), pl.ds(0, 128)],
                osem)
            pltpu.make_async_copy(
                tile_v,
                out_hbm.at[pl.ds(t0, _TC), pl.ds(0, 4), pl.ds(wid, 1),
                           pl.ds(0, 8), pl.ds(0, 128)],
                osem).wait()
            return carry

        lax.fori_loop(0, n_chunks, chunk, 0)

    return k(idx_t, table)


def kernel(inputs, embedding_matrix):
    batch, hist = inputs.shape
    idx_t = inputs.T.astype(jnp.int32)
    out5 = _sc_gather(idx_t, embedding_matrix, hist=hist)
    x = out5.transpose(2, 4, 0, 1, 3)
    return x.reshape(batch, hist, _EMBED_DIM)
